# baseline (speedup 1.0000x reference)
import functools

import jax
import jax.numpy as jnp
from jax.experimental import pallas as pl
from jax.experimental.pallas import tpu as pltpu

_EPS = 1e-10
_TOP_P = 0.5


def _body(x_ref, masks_ref, wt_ref, out_ref, part_ref):
    Kb = x_ref.shape[0]
    L = x_ref.shape[1]
    H = x_ref.shape[2]
    N = Kb * L * H

    x2d = x_ref[...].reshape(N, L)
    wt = wt_ref[...]

    lgT = jax.lax.dot_general(wt, x2d, (((1,), (1,)), ((), ())),
                              preferred_element_type=jnp.float32)
    c0 = lgT[0:1, :]
    c1 = lgT[1:2, :]
    c2 = lgT[2:3, :]

    cm = jnp.maximum(jnp.maximum(c0, c1), c2)
    e0 = jnp.exp(c0 - cm)
    e1 = jnp.exp(c1 - cm)
    e2 = jnp.exp(c2 - cm)
    s = e0 + e1 + e2
    p0 = e0 / s
    p1 = e1 / s
    p2 = e2 / s

    ge01 = p0 >= p1
    ge02 = p0 >= p2
    ge12 = p1 >= p2
    is0 = ge01 & ge02
    is1 = (~is0) & ge12
    is2 = ~(is0 | is1)
    pmax = jnp.where(is0, p0, jnp.where(is1, p1, p2))
    sec0 = (is1 & ge02) | (is2 & ge01)
    sec1 = (is0 & ge12) | (is2 & ~ge01)
    sec2 = (is0 & ~ge12) | (is1 & ~ge02)
    psec = jnp.where(sec0, p0, jnp.where(sec1, p1, p2))

    keep2 = pmax <= _TOP_P
    k2f = keep2.astype(jnp.float32)
    g0 = is0.astype(jnp.float32) + k2f * sec0.astype(jnp.float32)
    g1 = is1.astype(jnp.float32) + k2f * sec1.astype(jnp.float32)
    g2 = is2.astype(jnp.float32) + k2f * sec2.astype(jnp.float32)

    ent = -(p0 * jnp.log(p0 + _EPS)
            + p1 * jnp.log(p1 + _EPS)
            + p2 * jnp.log(p2 + _EPS))

    part_ref[...] = jnp.concatenate([pmax, psec * k2f, ent], axis=0)[None]

    gpack = jnp.concatenate(
        [g0, g1, g2, jnp.zeros((5, N), jnp.float32)], axis=0)
    gcols = jnp.transpose(gpack)
    g0c = gcols[:, 0:1].reshape(Kb, L, H, 1)
    g1c = gcols[:, 1:2].reshape(Kb, L, H, 1)
    g2c = gcols[:, 2:3].reshape(Kb, L, H, 1)

    rows = jax.lax.broadcasted_iota(jnp.int32, (Kb, L, H, L), 1)
    cols = jax.lax.broadcasted_iota(jnp.int32, (Kb, L, H, L), 3)
    eye = (rows == cols).astype(jnp.float32)
    m0 = masks_ref[0][None]
    m1 = masks_ref[1][None]
    m2 = masks_ref[2][None]
    out_ref[...] = g0c * m0 + g1c * m1 + g2c * m2 + eye


def _loss_body(part_ref, loss_ref, *, H, L, BH):
    parts = part_ref[...]
    N = parts.shape[2]
    acc = jnp.sum(parts, axis=0)
    n_iota = jax.lax.broadcasted_iota(jnp.int32, (N, L), 0)
    l_iota = jax.lax.broadcasted_iota(jnp.int32, (N, L), 1)
    fold = (((n_iota // H) % L) == l_iota).astype(jnp.float32)
    imp = jnp.dot(acc[0:2, :], fold,
                  preferred_element_type=jnp.float32)
    n = jnp.float32(3 * L)
    mean = jnp.sum(imp) / n
    d = imp - mean
    ssq = jnp.sum(d * d) + jnp.float32(L) * mean * mean
    var = ssq / (n - 1.0)
    loss_imp = var / (mean * mean + _EPS)
    loss_dyn = jnp.sum(acc[2:3, :]) / jnp.float32(3 * BH)
    loss_ref[0] = loss_imp + 0.1 * loss_dyn


@functools.partial(jax.jit, static_argnames=())
def kernel(x, masks, W_gate):
    B, H, L, _ = x.shape
    xp = jnp.transpose(x, (0, 2, 1, 3))
    masks_t = jnp.transpose(masks, (1, 0, 2))
    masks_b = jnp.broadcast_to(
        masks_t[:, :, None, :], (3, L, H, L))
    w_t = jnp.transpose(W_gate)

    Kb = 4
    G = B // Kb
    N = Kb * L * H
    out, parts = pl.pallas_call(
        _body,
        grid=(G,),
        in_specs=[
            pl.BlockSpec((Kb, L, H, L), lambda i: (i, 0, 0, 0)),
            pl.BlockSpec((3, L, H, L), lambda i: (0, 0, 0, 0)),
            pl.BlockSpec((3, L), lambda i: (0, 0)),
        ],
        out_specs=[
            pl.BlockSpec((Kb, L, H, L), lambda i: (i, 0, 0, 0)),
            pl.BlockSpec((1, 3, N), lambda i: (i, 0, 0)),
        ],
        out_shape=[
            jax.ShapeDtypeStruct((B, L, H, L), jnp.float32),
            jax.ShapeDtypeStruct((G, 3, N), jnp.float32),
        ],
        compiler_params=pltpu.CompilerParams(
            dimension_semantics=("parallel",)),
    )(xp, masks_b, w_t)

    loss1 = pl.pallas_call(
        functools.partial(_loss_body, H=H, L=L, BH=B * H),
        in_specs=[pl.BlockSpec((G, 3, N), lambda: (0, 0, 0))],
        out_specs=pl.BlockSpec(memory_space=pltpu.SMEM),
        out_shape=jax.ShapeDtypeStruct((1,), jnp.float32),
    )(parts)

    return jnp.transpose(out, (0, 2, 1, 3)), loss1[0]


# speedup vs baseline: 1.0596x; 1.0596x over previous
import functools

import jax
import jax.numpy as jnp
from jax.experimental import pallas as pl
from jax.experimental.pallas import tpu as pltpu

_EPS = 1e-10
_TOP_P = 0.5


def _body(x_ref, masks_ref, wt_ref, out_ref, loss_ref, mb_ref, imp_acc,
          ent_acc):
    i = pl.program_id(0)
    nsteps = pl.num_programs(0)
    Kb = x_ref.shape[0]
    L = x_ref.shape[1]
    H = x_ref.shape[2]
    N = Kb * L * H

    @pl.when(i == 0)
    def _init():
        imp_acc[...] = jnp.zeros_like(imp_acc)
        ent_acc[0] = jnp.float32(0.0)
        for e in range(3):
            mb_ref[e] = jnp.broadcast_to(
                masks_ref[e][:, None, :], (L, H, masks_ref.shape[2]))

    x2d = x_ref[...].reshape(N, L)
    wt = wt_ref[...]

    lgT = jax.lax.dot_general(wt, x2d, (((1,), (1,)), ((), ())),
                              preferred_element_type=jnp.float32)
    c0 = lgT[0:1, :]
    c1 = lgT[1:2, :]
    c2 = lgT[2:3, :]

    cm = jnp.maximum(jnp.maximum(c0, c1), c2)
    e0 = jnp.exp(c0 - cm)
    e1 = jnp.exp(c1 - cm)
    e2 = jnp.exp(c2 - cm)
    s = e0 + e1 + e2
    p0 = e0 / s
    p1 = e1 / s
    p2 = e2 / s

    ge01 = p0 >= p1
    ge02 = p0 >= p2
    ge12 = p1 >= p2
    is0 = ge01 & ge02
    is1 = (~is0) & ge12
    is2 = ~(is0 | is1)
    pmax = jnp.where(is0, p0, jnp.where(is1, p1, p2))
    sec0 = (is1 & ge02) | (is2 & ge01)
    sec1 = (is0 & ge12) | (is2 & ~ge01)
    sec2 = (is0 & ~ge12) | (is1 & ~ge02)
    psec = jnp.where(sec0, p0, jnp.where(sec1, p1, p2))

    keep2 = pmax <= _TOP_P
    k2f = keep2.astype(jnp.float32)
    g0 = is0.astype(jnp.float32) + k2f * sec0.astype(jnp.float32)
    g1 = is1.astype(jnp.float32) + k2f * sec1.astype(jnp.float32)
    g2 = is2.astype(jnp.float32) + k2f * sec2.astype(jnp.float32)

    ent = -(p0 * jnp.log(p0 + _EPS)
            + p1 * jnp.log(p1 + _EPS)
            + p2 * jnp.log(p2 + _EPS))
    ent_acc[0] += jnp.sum(ent)

    imp_acc[0:1, :] += pmax
    imp_acc[1:2, :] += psec * k2f

    gpack = jnp.concatenate(
        [g0, g1, g2, jnp.zeros((5, N), jnp.float32)], axis=0)
    gcols = jnp.transpose(gpack)
    g0c = gcols[:, 0:1].reshape(Kb, L, H, 1)
    g1c = gcols[:, 1:2].reshape(Kb, L, H, 1)
    g2c = gcols[:, 2:3].reshape(Kb, L, H, 1)

    rows = jax.lax.broadcasted_iota(jnp.int32, (Kb, L, H, L), 1)
    cols = jax.lax.broadcasted_iota(jnp.int32, (Kb, L, H, L), 3)
    eye = (rows == cols).astype(jnp.float32)
    out_ref[...] = (g0c * mb_ref[0][None] + g1c * mb_ref[1][None]
                    + g2c * mb_ref[2][None] + eye)

    @pl.when(i == nsteps - 1)
    def _finalize():
        n_iota = jax.lax.broadcasted_iota(jnp.int32, (N, L), 0)
        l_iota = jax.lax.broadcasted_iota(jnp.int32, (N, L), 1)
        fold = (((n_iota // H) % L) == l_iota).astype(jnp.float32)
        imp = jnp.dot(imp_acc[...], fold,
                      preferred_element_type=jnp.float32)
        n = jnp.float32(3 * L)
        mean = jnp.sum(imp) / n
        d = imp - mean
        ssq = jnp.sum(d * d) + jnp.float32(L) * mean * mean
        var = ssq / (n - 1.0)
        loss_imp = var / (mean * mean + _EPS)
        loss_dyn = ent_acc[0] / jnp.float32(3 * 512)
        loss_ref[0] = loss_imp + 0.1 * loss_dyn


@functools.partial(jax.jit, static_argnames=())
def kernel(x, masks, W_gate):
    B, H, L, _ = x.shape
    xp = jnp.transpose(x, (0, 2, 1, 3))
    masks_t = jnp.transpose(masks, (1, 0, 2))
    w_t = jnp.transpose(W_gate)

    Kb = 4
    out, loss1 = pl.pallas_call(
        _body,
        grid=(B // Kb,),
        in_specs=[
            pl.BlockSpec((Kb, L, H, L), lambda i: (i, 0, 0, 0)),
            pl.BlockSpec((3, L, L), lambda i: (0, 0, 0)),
            pl.BlockSpec((3, L), lambda i: (0, 0)),
        ],
        out_specs=[
            pl.BlockSpec((Kb, L, H, L), lambda i: (i, 0, 0, 0)),
            pl.BlockSpec(memory_space=pltpu.SMEM),
        ],
        out_shape=[
            jax.ShapeDtypeStruct((B, L, H, L), jnp.float32),
            jax.ShapeDtypeStruct((1,), jnp.float32),
        ],
        scratch_shapes=[
            pltpu.VMEM((3, L, H, L), jnp.float32),
            pltpu.VMEM((2, Kb * L * H), jnp.float32),
            pltpu.SMEM((1,), jnp.float32),
        ],
    )(xp, masks_t, w_t)

    return jnp.transpose(out, (0, 2, 1, 3)), loss1[0]
